# baseline (speedup 1.0000x reference)
import jax
import jax.numpy as jnp
from jax import lax
from jax.experimental import pallas as pl
from jax.experimental.pallas import tpu as pltpu
from jax.experimental.pallas import tpu_sc as plsc

BATCH = 4096
HIST = 50
D_DIM = 64
I_DIM = 1000000
HPAD = 64

NC = 2
NS = 16
NW = NC * NS
B_W = BATCH // NW
NBUF = 4
N_OUTER = B_W // NBUF


def _gather_body(x_hbm, tab_hbm, out_hbm, idx_v, rows_a, rows_b, rows_c, rows_d,
                 gsem, osem):
    cid = lax.axis_index("c")
    sid = lax.axis_index("s")
    wid = sid * NC + cid
    rows_bufs = (rows_a, rows_b, rows_c, rows_d)

    pltpu.sync_copy(x_hbm.at[pl.ds(wid * B_W, B_W), :], idx_v)

    def gather_start(bb, b):
        for k16 in range(HIST // 16 + 1):
            v = idx_v[bb, pl.ds(k16 * 16, 16)]
            for j in range(16):
                k = k16 * 16 + j
                if k < HIST:
                    i = v[j]
                    pltpu.make_async_copy(
                        tab_hbm.at[i >> 4, (i >> 1) & 7,
                                   pl.ds((i & 1) * D_DIM, D_DIM)],
                        rows_bufs[b].at[k],
                        gsem.at[b],
                    ).start()

    def gather_wait(b):
        def row(k, _):
            pltpu.make_async_copy(
                tab_hbm.at[0, 0, pl.ds(0, D_DIM)], rows_bufs[b].at[0], gsem.at[b]
            ).wait()
            return ()

        lax.fori_loop(0, HIST, row, ())

    def out_start(bb, b):
        pltpu.make_async_copy(
            rows_bufs[b], out_hbm.at[wid * B_W + bb], osem.at[b]
        ).start()

    def out_wait(b):
        pltpu.make_async_copy(
            rows_bufs[b], out_hbm.at[0], osem.at[b]
        ).wait()

    for b in range(NBUF):
        gather_start(b, b)

    def outer(g, _):
        for b in range(NBUF):
            bb = g * NBUF + b
            gather_wait(b)
            out_start(bb, b)
            out_wait(b)

            @pl.when(bb + NBUF < B_W)
            def _():
                gather_start(bb + NBUF, b)

        return ()

    lax.fori_loop(0, N_OUTER, outer, ())


def kernel(x, item_emb_mat):
    tab3 = item_emb_mat[:I_DIM].reshape(I_DIM // 16, 8, 2 * D_DIM)
    xpad = jnp.pad(x.astype(jnp.int32), ((0, 0), (0, HPAD - HIST)))
    mesh = plsc.VectorSubcoreMesh(core_axis_name="c", subcore_axis_name="s")
    out = pl.kernel(
        _gather_body,
        out_type=jax.ShapeDtypeStruct((BATCH, HIST, D_DIM), jnp.float32),
        mesh=mesh,
        compiler_params=pltpu.CompilerParams(use_tc_tiling_on_sc=True),
        scratch_types=[
            pltpu.VMEM((B_W, HPAD), jnp.int32),
            pltpu.VMEM((HIST, D_DIM), jnp.float32),
            pltpu.VMEM((HIST, D_DIM), jnp.float32),
            pltpu.VMEM((HIST, D_DIM), jnp.float32),
            pltpu.VMEM((HIST, D_DIM), jnp.float32),
            pltpu.SemaphoreType.DMA((NBUF,)),
            pltpu.SemaphoreType.DMA((NBUF,)),
        ],
    )(xpad, tab3)
    return out


# speedup vs baseline: 1.9342x; 1.9342x over previous
import jax
import jax.numpy as jnp
from jax import lax
from jax.experimental import pallas as pl
from jax.experimental.pallas import tpu as pltpu
from jax.experimental.pallas import tpu_sc as plsc

BATCH = 4096
HIST = 50
D_DIM = 64
I_DIM = 1000000
HPAD = 64

NC = 2
NS = 16
NW = NC * NS
B_W = BATCH // NW
NBUF = 4
N_OUTER = B_W // NBUF


def _gather_body(x_hbm, tab_hbm, out_hbm, idx_v, rows_a, rows_b, rows_c, rows_d,
                 gsem, osem):
    cid = lax.axis_index("c")
    sid = lax.axis_index("s")
    wid = sid * NC + cid
    rows_bufs = (rows_a, rows_b, rows_c, rows_d)

    pltpu.sync_copy(x_hbm.at[pl.ds(wid * B_W, B_W), :], idx_v)

    def gather_start(bb, b):
        for k16 in range(HIST // 16 + 1):
            v = idx_v[bb, pl.ds(k16 * 16, 16)]
            for j in range(16):
                k = k16 * 16 + j
                if k < HIST:
                    i = v[j]
                    pltpu.make_async_copy(
                        tab_hbm.at[i >> 3, i & 7], rows_bufs[b].at[k], gsem.at[b]
                    ).start()

    def gather_wait(b):
        def row(k, _):
            pltpu.make_async_copy(
                tab_hbm.at[0, 0], rows_bufs[b].at[0], gsem.at[b]
            ).wait()
            return ()

        lax.fori_loop(0, HIST, row, ())

    def out_start(bb, b):
        pltpu.make_async_copy(
            rows_bufs[b], out_hbm.at[wid * B_W + bb], osem.at[b]
        ).start()

    def out_wait(b):
        pltpu.make_async_copy(
            rows_bufs[b], out_hbm.at[0], osem.at[b]
        ).wait()

    for b in range(NBUF):
        gather_start(b, b)

    def outer(g, _):
        for b in range(NBUF):
            bb = g * NBUF + b
            gather_wait(b)
            out_start(bb, b)
            out_wait(b)

            @pl.when(bb + NBUF < B_W)
            def _():
                gather_start(bb + NBUF, b)

        return ()

    lax.fori_loop(0, N_OUTER, outer, ())


def kernel(x, item_emb_mat):
    tab3 = item_emb_mat[:I_DIM].reshape(I_DIM // 8, 8, D_DIM)
    xpad = jnp.pad(x.astype(jnp.int32), ((0, 0), (0, HPAD - HIST)))
    mesh = plsc.VectorSubcoreMesh(core_axis_name="c", subcore_axis_name="s")
    out = pl.kernel(
        _gather_body,
        out_type=jax.ShapeDtypeStruct((BATCH, HIST, D_DIM), jnp.float32),
        mesh=mesh,
        compiler_params=pltpu.CompilerParams(use_tc_tiling_on_sc=True),
        scratch_types=[
            pltpu.VMEM((B_W, HPAD), jnp.int32),
            pltpu.VMEM((HIST, D_DIM), jnp.float32),
            pltpu.VMEM((HIST, D_DIM), jnp.float32),
            pltpu.VMEM((HIST, D_DIM), jnp.float32),
            pltpu.VMEM((HIST, D_DIM), jnp.float32),
            pltpu.SemaphoreType.DMA((NBUF,)),
            pltpu.SemaphoreType.DMA((NBUF,)),
        ],
    )(xpad, tab3)
    return out
